# baseline (speedup 1.0000x reference)
import jax
import jax.numpy as jnp
from jax.experimental import pallas as pl
from jax.experimental.pallas import tpu as pltpu

_R1 = 28 * 32
_R2 = 12 * 64
_S1 = 936
_S2 = 840
_G = 32
_FEAT = 48 * 16


def _rup(a, b):
    return (a + b - 1) // b * b


def _conv_body(x_ref, w1_ref, b1_ref, w2_ref, b2_ref, sel_ref, o_ref,
               sc1, scw, sc2, xb, swb):
    f32 = jnp.float32
    bf16 = jnp.bfloat16
    for kh in range(5):
        xb[:, 512 * kh:512 * (kh + 1)] = x_ref[pl.ds(32 * kh, _R1), :]
    a1 = jnp.dot(xb[...], w1_ref[...], preferred_element_type=f32)
    a1 = jnp.maximum(a1 + b1_ref[...], 0.0)

    sc1[pl.ds(_R1, _S1 - _R1), :] = jnp.zeros((_S1 - _R1, 256), bf16)
    sc1[pl.ds(0, _R1), :] = a1.astype(bf16)
    p1 = jnp.maximum(
        jnp.maximum(sc1[pl.ds(0, _R1), :], sc1[pl.ds(1, _R1), :]),
        jnp.maximum(sc1[pl.ds(32, _R1), :], sc1[pl.ds(33, _R1), :]))
    sc1[pl.ds(0, _R1), :] = p1

    for s in range(3):
        scw[pl.ds(0, _R1), 256 * s:256 * (s + 1)] = sc1[pl.ds(2 * s, _R1), :]
    for kh in range(3):
        swb[:, 768 * kh:768 * (kh + 1)] = scw[pl.ds(64 * kh, _R2), :]
    a2 = jnp.dot(swb[...], w2_ref[...], preferred_element_type=f32)
    a2 = jnp.maximum(a2 + b2_ref[...], 0.0)

    sc2[pl.ds(_R2, _S2 - _R2), :] = jnp.zeros((_S2 - _R2, 512), bf16)
    sc2[pl.ds(0, _R2), :] = a2.astype(bf16)
    p2 = jnp.maximum(
        jnp.maximum(sc2[pl.ds(0, _R2), :], sc2[pl.ds(2, _R2), :]),
        jnp.maximum(sc2[pl.ds(64, _R2), :], sc2[pl.ds(66, _R2), :]))

    o_ref[...] = jnp.dot(sel_ref[...], p2, preferred_element_type=f32)


def _mlp_body(x_ref, w1_ref, b1_ref, w2_ref, b2_ref, w3_ref, b3_ref, o_ref):
    h = jnp.dot(x_ref[...], w1_ref[...], preferred_element_type=jnp.float32)
    h = jnp.maximum(h + b1_ref[...], 0.0)
    h = jnp.dot(h, w2_ref[...], preferred_element_type=jnp.float32)
    h = jnp.maximum(h + b2_ref[...], 0.0)
    h = jnp.dot(h, w3_ref[...], preferred_element_type=jnp.float32)
    o_ref[...] = h + b3_ref[...]


def _forward(params, x):
    n = x.shape[0]
    npad = _rup(n, _G)
    if npad != n:
        x = jnp.pad(x, ((0, npad - n), (0, 0), (0, 0), (0, 0)))
    ng = npad // _G

    xc = jnp.transpose(x.reshape(ng, _G, 3, 1024), (0, 3, 2, 1))
    xc = jnp.pad(xc.reshape(ng, 1024, 96).astype(jnp.bfloat16),
                 ((0, 0), (0, 8), (0, 0)))
    xw = jnp.concatenate([xc[:, kw:kw + 1024, :] for kw in range(5)], axis=2)
    xw = jnp.pad(xw, ((0, 0), (0, 0), (0, 32)))

    feats = pl.pallas_call(
        _conv_body,
        out_shape=jax.ShapeDtypeStruct((ng, 48, 512), jnp.float32),
        grid=(ng,),
        in_specs=[
            pl.BlockSpec((None, 1024, 512), lambda i: (i, 0, 0)),
            pl.BlockSpec((2560, 256), lambda i: (0, 0)),
            pl.BlockSpec((1, 256), lambda i: (0, 0)),
            pl.BlockSpec((2304, 512), lambda i: (0, 0)),
            pl.BlockSpec((1, 512), lambda i: (0, 0)),
            pl.BlockSpec((48, _R2), lambda i: (0, 0)),
        ],
        out_specs=pl.BlockSpec((None, 48, 512), lambda i: (i, 0, 0)),
        scratch_shapes=[pltpu.VMEM((_S1, 256), jnp.bfloat16),
                        pltpu.VMEM((_R1, 768), jnp.bfloat16),
                        pltpu.VMEM((_S2, 512), jnp.bfloat16),
                        pltpu.VMEM((_R1, 2560), jnp.bfloat16),
                        pltpu.VMEM((_R2, 2304), jnp.bfloat16)],
        compiler_params=pltpu.CompilerParams(
            dimension_semantics=("parallel",)),
    )(xw, params["w1"], params["b1"], params["w2"], params["b2"],
      params["sel"])

    feats = jnp.transpose(feats.reshape(ng, 48, _G, 16), (0, 2, 1, 3))
    feats = feats.reshape(npad, _FEAT)

    bt = min(256, _rup(npad, 8))
    mpad = _rup(npad, bt)
    if mpad != npad:
        feats = jnp.pad(feats, ((0, mpad - npad), (0, 0)))
    logits = pl.pallas_call(
        _mlp_body,
        out_shape=jax.ShapeDtypeStruct((mpad, 128), jnp.float32),
        grid=(mpad // bt,),
        in_specs=[
            pl.BlockSpec((bt, _FEAT), lambda i: (i, 0)),
            pl.BlockSpec((_FEAT, 128), lambda i: (0, 0)),
            pl.BlockSpec((1, 128), lambda i: (0, 0)),
            pl.BlockSpec((128, 128), lambda i: (0, 0)),
            pl.BlockSpec((1, 128), lambda i: (0, 0)),
            pl.BlockSpec((128, 128), lambda i: (0, 0)),
            pl.BlockSpec((1, 128), lambda i: (0, 0)),
        ],
        out_specs=pl.BlockSpec((bt, 128), lambda i: (i, 0)),
        compiler_params=pltpu.CompilerParams(
            dimension_semantics=("parallel",)),
    )(feats, params["fc1_w"], params["fc1_b"], params["fc2_w"],
      params["fc2_b"], params["fc3_w"], params["fc3_b"])
    return logits[:n, :10]


_forward_jit = jax.jit(_forward)


def _prep(conv1_w, conv1_b, conv2_w, conv2_b,
          fc1_w, fc1_b, fc2_w, fc2_b, fc3_w, fc3_b):
    f32 = jnp.float32
    bf16 = jnp.bfloat16
    eye = jnp.eye(_G, dtype=f32)
    t1 = jnp.transpose(conv1_w, (2, 3, 1, 0)).reshape(5, 15, 6)
    base1 = jnp.zeros((5, 16, 8), f32).at[:, :15, :6].set(t1)
    w1 = jnp.einsum('jk,tuv->tujkv', eye, base1).reshape(2560, 256)
    b1 = jnp.tile(jnp.zeros((8,), f32).at[:6].set(conv1_b), _G).reshape(1, 256)
    t2 = jnp.transpose(conv2_w, (2, 3, 1, 0))
    t2 = jnp.pad(t2, ((0, 0), (0, 0), (0, 2), (0, 0)))
    w2 = jnp.einsum('jk,hwcv->hwjckv', eye, t2).reshape(2304, 512)
    b2 = jnp.tile(conv2_b, _G).reshape(1, 512)
    r6 = jnp.arange(6)
    rows = (r6[:, None] * 8 + r6[None, :]).reshape(-1)
    cols = (r6[:, None] * 128 + 4 * r6[None, :]).reshape(-1)
    sel = jnp.zeros((48, _R2), f32).at[rows, cols].set(1.0)
    tf = jnp.transpose(fc1_w.reshape(128, 16, 6, 6), (2, 3, 1, 0))
    tf = jnp.pad(tf, ((0, 0), (0, 2), (0, 0), (0, 0)))
    return {
        "w1": w1.astype(bf16), "b1": b1, "w2": w2.astype(bf16), "b2": b2,
        "sel": sel.astype(bf16),
        "fc1_w": tf.reshape(_FEAT, 128),
        "fc1_b": fc1_b.reshape(1, 128),
        "fc2_w": jnp.zeros((128, 128), f32).at[:, :64].set(fc2_w.T),
        "fc2_b": jnp.zeros((1, 128), f32).at[0, :64].set(fc2_b),
        "fc3_w": jnp.zeros((128, 128), f32).at[:64, :10].set(fc3_w.T),
        "fc3_b": jnp.zeros((1, 128), f32).at[0, :10].set(fc3_b),
    }


def kernel(conv1_w, conv1_b, conv2_w, conv2_b,
           fc1_w, fc1_b, fc2_w, fc2_b, fc3_w, fc3_b, x):
    params = _prep(conv1_w, conv1_b, conv2_w, conv2_b,
                   fc1_w, fc1_b, fc2_w, fc2_b, fc3_w, fc3_b)
    return _forward_jit(params, x)


# speedup vs baseline: 1.1928x; 1.1928x over previous
import jax
import jax.numpy as jnp
from jax.experimental import pallas as pl
from jax.experimental.pallas import tpu as pltpu

_R1 = 28 * 32
_R2 = 12 * 64
_S1 = 936
_S2 = 840
_G = 32
_FEAT = 48 * 16


def _rup(a, b):
    return (a + b - 1) // b * b


def _conv_body(x_ref, w1_ref, b1_ref, w2_ref, b2_ref, sel_ref, o_ref,
               sc1, scw, sc2, xb, swb):
    f32 = jnp.float32
    bf16 = jnp.bfloat16
    for g in range(2):
        for kh in range(5):
            xb[g, :, 512 * kh:512 * (kh + 1)] = x_ref[g, pl.ds(32 * kh, _R1), :]
        a1 = jnp.dot(xb[g], w1_ref[...], preferred_element_type=f32)
        a1 = jnp.maximum(a1 + b1_ref[...], 0.0)

        sc1[g, pl.ds(_R1, _S1 - _R1), :] = jnp.zeros((_S1 - _R1, 256), bf16)
        sc1[g, pl.ds(0, _R1), :] = a1.astype(bf16)
        p1 = jnp.maximum(
            jnp.maximum(sc1[g, pl.ds(0, _R1), :], sc1[g, pl.ds(1, _R1), :]),
            jnp.maximum(sc1[g, pl.ds(32, _R1), :], sc1[g, pl.ds(33, _R1), :]))
        sc1[g, pl.ds(0, _R1), :] = p1

        for s in range(3):
            scw[g, pl.ds(0, _R1), 256 * s:256 * (s + 1)] = \
                sc1[g, pl.ds(2 * s, _R1), :]
        for kh in range(3):
            swb[g, :, 768 * kh:768 * (kh + 1)] = scw[g, pl.ds(64 * kh, _R2), :]
        a2 = jnp.dot(swb[g], w2_ref[...], preferred_element_type=f32)
        a2 = jnp.maximum(a2 + b2_ref[...], 0.0)

        sc2[g, pl.ds(_R2, _S2 - _R2), :] = jnp.zeros((_S2 - _R2, 512), bf16)
        sc2[g, pl.ds(0, _R2), :] = a2.astype(bf16)
        p2 = jnp.maximum(
            jnp.maximum(sc2[g, pl.ds(0, _R2), :], sc2[g, pl.ds(2, _R2), :]),
            jnp.maximum(sc2[g, pl.ds(64, _R2), :], sc2[g, pl.ds(66, _R2), :]))

        o_ref[g] = jnp.dot(sel_ref[...], p2,
                           preferred_element_type=f32).astype(bf16)


def _mlp_body(x_ref, w1_ref, b1_ref, w2_ref, b2_ref, w3_ref, b3_ref, o_ref):
    h = jnp.dot(x_ref[...], w1_ref[...], preferred_element_type=jnp.float32)
    h = jnp.maximum(h + b1_ref[...], 0.0)
    h = jnp.dot(h, w2_ref[...], preferred_element_type=jnp.float32)
    h = jnp.maximum(h + b2_ref[...], 0.0)
    h = jnp.dot(h, w3_ref[...], preferred_element_type=jnp.float32)
    o_ref[...] = h + b3_ref[...]


def _forward(params, x):
    n = x.shape[0]
    npad = _rup(n, 2 * _G)
    if npad != n:
        x = jnp.pad(x, ((0, npad - n), (0, 0), (0, 0), (0, 0)))
    ng = npad // _G

    xc = jnp.transpose(x.reshape(ng, _G, 3, 1024), (0, 3, 2, 1))
    xc = jnp.pad(xc.reshape(ng, 1024, 96).astype(jnp.bfloat16),
                 ((0, 0), (0, 8), (0, 0)))
    xw = jnp.concatenate([xc[:, kw:kw + 1024, :] for kw in range(5)], axis=2)
    xw = jnp.pad(xw, ((0, 0), (0, 0), (0, 32)))
    ng2 = ng // 2
    xw = xw.reshape(ng2, 2, 1024, 512)

    feats = pl.pallas_call(
        _conv_body,
        out_shape=jax.ShapeDtypeStruct((ng2, 2, 48, 512), jnp.bfloat16),
        grid=(ng2,),
        in_specs=[
            pl.BlockSpec((None, 2, 1024, 512), lambda i: (i, 0, 0, 0)),
            pl.BlockSpec((2560, 256), lambda i: (0, 0)),
            pl.BlockSpec((1, 256), lambda i: (0, 0)),
            pl.BlockSpec((2304, 512), lambda i: (0, 0)),
            pl.BlockSpec((1, 512), lambda i: (0, 0)),
            pl.BlockSpec((48, _R2), lambda i: (0, 0)),
        ],
        out_specs=pl.BlockSpec((None, 2, 48, 512), lambda i: (i, 0, 0, 0)),
        scratch_shapes=[pltpu.VMEM((2, _S1, 256), jnp.bfloat16),
                        pltpu.VMEM((2, _R1, 768), jnp.bfloat16),
                        pltpu.VMEM((2, _S2, 512), jnp.bfloat16),
                        pltpu.VMEM((2, _R1, 2560), jnp.bfloat16),
                        pltpu.VMEM((2, _R2, 2304), jnp.bfloat16)],
        compiler_params=pltpu.CompilerParams(
            dimension_semantics=("parallel",)),
    )(xw, params["w1"], params["b1"], params["w2"], params["b2"],
      params["sel"])

    feats = jnp.transpose(feats.reshape(ng, 48, _G, 16), (0, 2, 1, 3))
    feats = feats.reshape(npad, _FEAT)

    bt = min(512, _rup(npad, 8))
    mpad = _rup(npad, bt)
    if mpad != npad:
        feats = jnp.pad(feats, ((0, mpad - npad), (0, 0)))
    logits = pl.pallas_call(
        _mlp_body,
        out_shape=jax.ShapeDtypeStruct((mpad, 128), jnp.float32),
        grid=(mpad // bt,),
        in_specs=[
            pl.BlockSpec((bt, _FEAT), lambda i: (i, 0)),
            pl.BlockSpec((_FEAT, 128), lambda i: (0, 0)),
            pl.BlockSpec((1, 128), lambda i: (0, 0)),
            pl.BlockSpec((128, 128), lambda i: (0, 0)),
            pl.BlockSpec((1, 128), lambda i: (0, 0)),
            pl.BlockSpec((128, 128), lambda i: (0, 0)),
            pl.BlockSpec((1, 128), lambda i: (0, 0)),
        ],
        out_specs=pl.BlockSpec((bt, 128), lambda i: (i, 0)),
        compiler_params=pltpu.CompilerParams(
            dimension_semantics=("parallel",)),
    )(feats, params["fc1_w"], params["fc1_b"], params["fc2_w"],
      params["fc2_b"], params["fc3_w"], params["fc3_b"])
    return logits[:n, :10]


_forward_jit = jax.jit(_forward)


def _prep(conv1_w, conv1_b, conv2_w, conv2_b,
          fc1_w, fc1_b, fc2_w, fc2_b, fc3_w, fc3_b):
    f32 = jnp.float32
    bf16 = jnp.bfloat16
    eye = jnp.eye(_G, dtype=f32)
    t1 = jnp.transpose(conv1_w, (2, 3, 1, 0)).reshape(5, 15, 6)
    base1 = jnp.zeros((5, 16, 8), f32).at[:, :15, :6].set(t1)
    w1 = jnp.einsum('jk,tuv->tujkv', eye, base1).reshape(2560, 256)
    b1 = jnp.tile(jnp.zeros((8,), f32).at[:6].set(conv1_b), _G).reshape(1, 256)
    t2 = jnp.transpose(conv2_w, (2, 3, 1, 0))
    t2 = jnp.pad(t2, ((0, 0), (0, 0), (0, 2), (0, 0)))
    w2 = jnp.einsum('jk,hwcv->hwjckv', eye, t2).reshape(2304, 512)
    b2 = jnp.tile(conv2_b, _G).reshape(1, 512)
    r6 = jnp.arange(6)
    rows = (r6[:, None] * 8 + r6[None, :]).reshape(-1)
    cols = (r6[:, None] * 128 + 4 * r6[None, :]).reshape(-1)
    sel = jnp.zeros((48, _R2), f32).at[rows, cols].set(1.0)
    tf = jnp.transpose(fc1_w.reshape(128, 16, 6, 6), (2, 3, 1, 0))
    tf = jnp.pad(tf, ((0, 0), (0, 2), (0, 0), (0, 0)))
    return {
        "w1": w1.astype(bf16), "b1": b1, "w2": w2.astype(bf16), "b2": b2,
        "sel": sel.astype(bf16),
        "fc1_w": tf.reshape(_FEAT, 128).astype(bf16),
        "fc1_b": fc1_b.reshape(1, 128),
        "fc2_w": jnp.zeros((128, 128), f32).at[:, :64].set(fc2_w.T),
        "fc2_b": jnp.zeros((1, 128), f32).at[0, :64].set(fc2_b),
        "fc3_w": jnp.zeros((128, 128), f32).at[:64, :10].set(fc3_w.T),
        "fc3_b": jnp.zeros((1, 128), f32).at[0, :10].set(fc3_b),
    }


def kernel(conv1_w, conv1_b, conv2_w, conv2_b,
           fc1_w, fc1_b, fc2_w, fc2_b, fc3_w, fc3_b, x):
    params = _prep(conv1_w, conv1_b, conv2_w, conv2_b,
                   fc1_w, fc1_b, fc2_w, fc2_b, fc3_w, fc3_b)
    return _forward_jit(params, x)
